# baseline (speedup 1.0000x reference)
import functools

import jax
import jax.numpy as jnp
from jax import lax
from jax.experimental import pallas as pl
from jax.experimental.pallas import tpu as pltpu
from jax.experimental.pallas import tpu_sc as plsc

B, T, D = 16, 2048, 1024
NC, NS, L = 2, 16, 16
BPC = B // NC
CH = 32
NCHUNK = T // CH
KMAX = NCHUNK // NS
NSLOT = BPC * KMAX
NJ = D // L
ROT = 3


def _sc_body(payload, seq_lens, out, buf0, buf1, acc, lens_v, tmp, res,
             shared, sem0, sem1):
    c = lax.axis_index("c")
    s = lax.axis_index("s")
    bufs = (buf0, buf1)
    sems = (sem0, sem1)

    pltpu.sync_copy(seq_lens, lens_v.at[pl.ds(0, B)])

    def len_of(b):
        return lens_v[pl.ds(b, L)][0]

    zero = jnp.zeros((L,), jnp.float32)

    def zero_body(i, carry):
        for j in range(NJ):
            acc[i, pl.ds(j * L, L)] = zero
        return carry

    lax.fori_loop(0, BPC, zero_body, 0)

    def decode(slot):
        i = slot // KMAX
        k = slot % KMAX
        b = BPC * c + i
        phase = (s + ROT * i) % NS
        row0 = CH * (phase + NS * k)
        return i, b, row0, len_of(b)

    def start(slot, p, guard_last):
        i, b, row0, lb = decode(slot)
        cond = row0 < lb
        if guard_last:
            cond = cond & (slot < NSLOT)

        @pl.when(cond)
        def _():
            pltpu.make_async_copy(
                payload.at[b, pl.ds(row0, CH), :], bufs[p], sems[p]).start()

    def process(slot, p):
        i, b, row0, lb = decode(slot)

        @pl.when(row0 < lb)
        def _():
            pltpu.make_async_copy(
                payload.at[b, pl.ds(row0, CH), :], bufs[p], sems[p]).wait()
            nv = jnp.minimum(CH, lb - row0)
            buf = bufs[p]

            def row_body(r, carry):
                for j in range(NJ):
                    sl = pl.ds(j * L, L)
                    plsc.addupdate(acc.at[i, sl], buf[r, sl])
                return carry

            lax.fori_loop(0, nv, row_body, 0)

    start(0, 0, False)

    def main_body(idx, carry):
        m = idx * 2
        for pp in range(2):
            slot = m + pp
            start(slot + 1, (pp + 1) % 2, pp == 1)
            process(slot, pp)
        return carry

    lax.fori_loop(0, NSLOT // 2, main_body, 0)

    pltpu.sync_copy(acc, shared.at[s])
    plsc.subcore_barrier()

    @pl.when(s < BPC)
    def _():
        b = BPC * c + s
        pltpu.sync_copy(shared.at[0, s], res)

        def comb_body(w, carry):
            pltpu.sync_copy(shared.at[w, s], tmp)
            for j in range(NJ):
                sl = pl.ds(j * L, L)
                plsc.addupdate(res.at[sl], tmp[sl])
            return carry

        lax.fori_loop(1, NS, comb_body, 0)

        lb = len_of(b)
        recip = 1.0 / jnp.full((L,), lb).astype(jnp.float32)
        for j in range(NJ):
            sl = pl.ds(j * L, L)
            res[sl] = res[sl] * recip
        pltpu.sync_copy(res, out.at[b])


_sc_call = pl.kernel(
    _sc_body,
    out_type=jax.ShapeDtypeStruct((B, D), jnp.float32),
    mesh=plsc.VectorSubcoreMesh(core_axis_name="c", subcore_axis_name="s",
                                num_cores=NC, num_subcores=NS),
    scratch_types=[
        pltpu.VMEM((CH, D), jnp.float32),
        pltpu.VMEM((CH, D), jnp.float32),
        pltpu.VMEM((BPC, D), jnp.float32),
        pltpu.VMEM((B + L,), jnp.int32),
        pltpu.VMEM((D,), jnp.float32),
        pltpu.VMEM((D,), jnp.float32),
        pltpu.VMEM_SHARED((NS, BPC, D), jnp.float32),
        pltpu.SemaphoreType.DMA,
        pltpu.SemaphoreType.DMA,
    ],
)


@jax.jit
def kernel(payload, seq_lens):
    return _sc_call(payload, seq_lens.astype(jnp.int32))


# speedup vs baseline: 1.9833x; 1.9833x over previous
import functools

import jax
import jax.numpy as jnp
from jax import lax
from jax.experimental import pallas as pl
from jax.experimental.pallas import tpu as pltpu
from jax.experimental.pallas import tpu_sc as plsc

B, T, D = 16, 2048, 1024
NC, NS, L = 2, 16, 16
BPC = B // NC
CH = 16
NBUF = 4
NCHUNK = T // CH
KMAX = NCHUNK // NS
NSLOT = BPC * KMAX
NJ = D // L
ROT = 3


def _sc_body(payload, seq_lens, out, buf0, buf1, buf2, buf3, acc, lens_v,
             tmp, res, shared, sem0, sem1, sem2, sem3):
    c = lax.axis_index("c")
    s = lax.axis_index("s")
    bufs = (buf0, buf1, buf2, buf3)
    sems = (sem0, sem1, sem2, sem3)

    pltpu.sync_copy(seq_lens, lens_v.at[pl.ds(0, B)])

    def len_of(b):
        return lens_v[pl.ds(b, L)][0]

    zero = jnp.zeros((L,), jnp.float32)

    def zero_body(i, carry):
        for j in range(NJ):
            acc[i, pl.ds(j * L, L)] = zero
        return carry

    lax.fori_loop(0, BPC, zero_body, 0)

    def decode(slot):
        i = slot // KMAX
        k = slot % KMAX
        b = BPC * c + i
        phase = (s + ROT * i) % NS
        row0 = CH * (phase + NS * k)
        return i, b, row0, len_of(b)

    def start(slot, p, guard_last):
        i, b, row0, lb = decode(slot)
        cond = row0 < lb
        if guard_last:
            cond = cond & (slot < NSLOT)

        @pl.when(cond)
        def _():
            pltpu.make_async_copy(
                payload.at[b, pl.ds(row0, CH), :], bufs[p], sems[p]).start()

    def process(slot, p):
        i, b, row0, lb = decode(slot)

        @pl.when(row0 < lb)
        def _():
            pltpu.make_async_copy(
                payload.at[b, pl.ds(row0, CH), :], bufs[p], sems[p]).wait()
            nv = jnp.minimum(CH, lb - row0)
            buf = bufs[p]

            @pl.when(nv == CH)
            def _full():
                def col_body(j, carry):
                    sl = pl.ds(j * L, L)
                    vs = [buf[r, sl] for r in range(CH)]
                    while len(vs) > 1:
                        vs = [a + bb for a, bb in zip(vs[::2], vs[1::2])]
                    plsc.addupdate(acc.at[i, sl], vs[0])
                    return carry

                lax.fori_loop(0, NJ, col_body, 0)

            @pl.when(nv < CH)
            def _partial():
                def row_body(r, carry):
                    def col_body(j, carry2):
                        sl = pl.ds(j * L, L)
                        plsc.addupdate(acc.at[i, sl], buf[r, sl])
                        return carry2

                    lax.fori_loop(0, NJ, col_body, 0)
                    return carry

                lax.fori_loop(0, nv, row_body, 0)

    for p in range(NBUF - 1):
        start(p, p, False)

    def main_body(idx, carry):
        m = idx * NBUF
        for pp in range(NBUF):
            slot = m + pp
            start(slot + NBUF - 1, (pp + NBUF - 1) % NBUF, True)
            process(slot, pp)
        return carry

    lax.fori_loop(0, NSLOT // NBUF, main_body, 0)

    pltpu.sync_copy(acc, shared.at[s])
    plsc.subcore_barrier()

    @pl.when(s < BPC)
    def _():
        b = BPC * c + s
        pltpu.sync_copy(shared.at[0, s], res)

        def comb_body(w, carry):
            pltpu.sync_copy(shared.at[w, s], tmp)
            for j in range(NJ):
                sl = pl.ds(j * L, L)
                plsc.addupdate(res.at[sl], tmp[sl])
            return carry

        lax.fori_loop(1, NS, comb_body, 0)

        lb = len_of(b)
        recip = 1.0 / jnp.full((L,), lb).astype(jnp.float32)
        for j in range(NJ):
            sl = pl.ds(j * L, L)
            res[sl] = res[sl] * recip
        pltpu.sync_copy(res, out.at[b])


_sc_call = pl.kernel(
    _sc_body,
    out_type=jax.ShapeDtypeStruct((B, D), jnp.float32),
    mesh=plsc.VectorSubcoreMesh(core_axis_name="c", subcore_axis_name="s",
                                num_cores=NC, num_subcores=NS),
    scratch_types=[
        pltpu.VMEM((CH, D), jnp.float32),
        pltpu.VMEM((CH, D), jnp.float32),
        pltpu.VMEM((CH, D), jnp.float32),
        pltpu.VMEM((CH, D), jnp.float32),
        pltpu.VMEM((BPC, D), jnp.float32),
        pltpu.VMEM((B + L,), jnp.int32),
        pltpu.VMEM((D,), jnp.float32),
        pltpu.VMEM((D,), jnp.float32),
        pltpu.VMEM_SHARED((NS, BPC, D), jnp.float32),
        pltpu.SemaphoreType.DMA,
        pltpu.SemaphoreType.DMA,
        pltpu.SemaphoreType.DMA,
        pltpu.SemaphoreType.DMA,
    ],
)


@jax.jit
def kernel(payload, seq_lens):
    return _sc_call(payload, seq_lens.astype(jnp.int32))


# speedup vs baseline: 2.3529x; 1.1863x over previous
import functools

import jax
import jax.numpy as jnp
from jax import lax
from jax.experimental import pallas as pl
from jax.experimental.pallas import tpu as pltpu
from jax.experimental.pallas import tpu_sc as plsc

B, T, D = 16, 2048, 1024
NC, NS, L = 2, 16, 16
BPC = B // NC
CH = 32
NBUF = 3
NCHUNK = T // CH
KMAX = NCHUNK // NS
NSLOT = BPC * KMAX
NJ = D // L
ROT = 3


def _sc_body(payload, seq_lens, out, buf0, buf1, buf2, acc, lens_v,
             tmp, res, shared, sem0, sem1, sem2):
    c = lax.axis_index("c")
    s = lax.axis_index("s")
    bufs = (buf0, buf1, buf2)
    sems = (sem0, sem1, sem2)

    pltpu.sync_copy(seq_lens, lens_v.at[pl.ds(0, B)])

    def len_of(b):
        return lens_v[pl.ds(b, L)][0]

    zero = jnp.zeros((L,), jnp.float32)

    def zero_body(i, carry):
        for j in range(NJ):
            acc[i, pl.ds(j * L, L)] = zero
        return carry

    lax.fori_loop(0, BPC, zero_body, 0)

    def decode(slot):
        i = slot // KMAX
        k = slot % KMAX
        b = BPC * c + i
        phase = (s + ROT * i) % NS
        row0 = CH * (phase + NS * k)
        return i, b, row0, len_of(b)

    def start(slot, p):
        i, b, row0, lb = decode(slot)

        @pl.when(row0 < lb)
        def _():
            pltpu.make_async_copy(
                payload.at[b, pl.ds(row0, CH), :], bufs[p], sems[p]).start()

    def process(slot, p):
        i, b, row0, lb = decode(slot)

        @pl.when(row0 < lb)
        def _():
            pltpu.make_async_copy(
                payload.at[b, pl.ds(row0, CH), :], bufs[p], sems[p]).wait()
            nv = jnp.minimum(CH, lb - row0)
            buf = bufs[p]

            @pl.when(nv == CH)
            def _full():
                @plsc.parallel_loop(0, NJ, unroll=2)
                def _cols(j):
                    sl = pl.ds(j * L, L)
                    vs = [buf[r, sl] for r in range(CH)]
                    while len(vs) > 1:
                        vs = [a + bb for a, bb in zip(vs[::2], vs[1::2])]
                    plsc.addupdate(acc.at[i, sl], vs[0])

            @pl.when(nv < CH)
            def _partial():
                def row_body(r, carry):
                    @plsc.parallel_loop(0, NJ, unroll=4)
                    def _cols(j):
                        sl = pl.ds(j * L, L)
                        plsc.addupdate(acc.at[i, sl], buf[r, sl])

                    return carry

                lax.fori_loop(0, nv, row_body, 0)

    for p in range(NBUF - 1):
        start(p, p)

    ngroups = (NSLOT - (NBUF - 1)) // NBUF

    def main_body(g, carry):
        m = g * NBUF
        for pp in range(NBUF):
            slot = m + pp
            start(slot + NBUF - 1, (pp + NBUF - 1) % NBUF)
            process(slot, pp)
        return carry

    lax.fori_loop(0, ngroups, main_body, 0)
    for slot in range(ngroups * NBUF, NSLOT):
        process(slot, slot % NBUF)

    pltpu.sync_copy(acc, shared.at[s])
    plsc.subcore_barrier()

    @pl.when(s < BPC)
    def _():
        b = BPC * c + s
        pltpu.sync_copy(shared.at[0, s], res)

        def comb_body(w, carry):
            pltpu.sync_copy(shared.at[w, s], tmp)

            @plsc.parallel_loop(0, NJ, unroll=4)
            def _cols(j):
                sl = pl.ds(j * L, L)
                plsc.addupdate(res.at[sl], tmp[sl])

            return carry

        lax.fori_loop(1, NS, comb_body, 0)

        lb = len_of(b)
        recip = 1.0 / jnp.full((L,), lb).astype(jnp.float32)
        for j in range(NJ):
            sl = pl.ds(j * L, L)
            res[sl] = res[sl] * recip
        pltpu.sync_copy(res, out.at[b])


_sc_call = pl.kernel(
    _sc_body,
    out_type=jax.ShapeDtypeStruct((B, D), jnp.float32),
    mesh=plsc.VectorSubcoreMesh(core_axis_name="c", subcore_axis_name="s",
                                num_cores=NC, num_subcores=NS),
    scratch_types=[
        pltpu.VMEM((CH, D), jnp.float32),
        pltpu.VMEM((CH, D), jnp.float32),
        pltpu.VMEM((CH, D), jnp.float32),
        pltpu.VMEM((BPC, D), jnp.float32),
        pltpu.VMEM((B + L,), jnp.int32),
        pltpu.VMEM((D,), jnp.float32),
        pltpu.VMEM((D,), jnp.float32),
        pltpu.VMEM_SHARED((NS, BPC, D), jnp.float32),
        pltpu.SemaphoreType.DMA,
        pltpu.SemaphoreType.DMA,
        pltpu.SemaphoreType.DMA,
    ],
)


@jax.jit
def kernel(payload, seq_lens):
    return _sc_call(payload, seq_lens.astype(jnp.int32))
